# Initial kernel scaffold; baseline (speedup 1.0000x reference)
import jax
import jax.numpy as jnp
from jax.experimental import pallas as pl


def kernel(x, edge_index, batch, W, att_src, att_dst, bias):
    raise NotImplementedError("write your pallas kernel here")


# speedup vs baseline: 24.4668x; 24.4668x over previous
import functools

import jax
import jax.numpy as jnp
from jax import lax
from jax.experimental import pallas as pl
from jax.experimental.pallas import tpu as pltpu
from jax.experimental.pallas import tpu_sc as plsc

N = 10000
E = 320000
F = 128
H = 4
C = 32
G = 64

NW = 32
E2 = 331776
PER_W = E2 // NW
CH = 128
NCHUNK = PER_W // CH

BE = 512
NBLK = E2 // BE
NPAD = 10752
RW = 640
BN = 1000


def _proj_body(x_ref, w_ref, a_ref, h_ref, sn_ref):
    hb = jnp.dot(x_ref[...], w_ref[...], preferred_element_type=jnp.float32)
    h_ref[...] = hb
    sn_ref[...] = jnp.dot(hb, a_ref[...], preferred_element_type=jnp.float32)


def _project(x, W, Apack):
    return pl.pallas_call(
        _proj_body,
        grid=(N // BN,),
        in_specs=[
            pl.BlockSpec((BN, F), lambda b: (b, 0)),
            pl.BlockSpec((F, H * C), lambda b: (0, 0)),
            pl.BlockSpec((F, 16), lambda b: (0, 0)),
        ],
        out_specs=[
            pl.BlockSpec((BN, H * C), lambda b: (b, 0)),
            pl.BlockSpec((BN, 16), lambda b: (b, 0)),
        ],
        out_shape=[
            jax.ShapeDtypeStruct((N, H * C), jnp.float32),
            jax.ShapeDtypeStruct((N, 16), jnp.float32),
        ],
    )(x, W, Apack)


_sc_mesh = plsc.VectorSubcoreMesh(core_axis_name="c", subcore_axis_name="s")


@functools.partial(
    pl.kernel,
    mesh=_sc_mesh,
    out_type=jax.ShapeDtypeStruct((E2, F), jnp.float32),
    scratch_types=(
        pltpu.VMEM((CH,), jnp.int32),
        pltpu.VMEM((CH, F), jnp.float32),
        pltpu.SemaphoreType.DMA,
    ),
)
def _sc_gather(h_hbm, src_hbm, hsrc_o, src_v, hrows, sem1):
    wid = lax.axis_index("s") * 2 + lax.axis_index("c")
    base = wid * PER_W

    def body(i, carry):
        off = base + i * CH
        pltpu.sync_copy(src_hbm.at[pl.ds(off, CH)], src_v)
        pltpu.async_copy(h_hbm.at[src_v], hrows, sem1).wait()
        pltpu.sync_copy(hrows, hsrc_o.at[pl.ds(off, CH)])
        return carry

    lax.fori_loop(0, NCHUNK, body, 0)


def _seg_body(hsrc_ref, dstv_ref, snode_ref, apack_ref, dmin_ref, num_ref):
    b = pl.program_id(0)

    @pl.when(b == 0)
    def _():
        num_ref[...] = jnp.zeros_like(num_ref)

    dmin = dmin_ref[b, 0]
    dbase = pl.multiple_of((dmin // 8) * 8, 8)
    loc = dstv_ref[...] - dbase
    iota = lax.broadcasted_iota(jnp.int32, (BE, RW), 1)
    oh = (loc == iota).astype(jnp.float32)

    hs = hsrc_ref[...]
    asrc_e = jnp.dot(hs, apack_ref[...],
                     preferred_element_type=jnp.float32)
    snb = snode_ref[pl.ds(dbase, RW), :]
    adst_e = jnp.dot(oh, snb, preferred_element_type=jnp.float32)

    e = asrc_e[:, 0:H] + adst_e[:, H:2 * H]
    e = jnp.where(e > 0, e, 0.2 * e)
    s = jnp.exp(e)
    parts = [s[:, hh:hh + 1] * hs[:, hh * C:(hh + 1) * C] for hh in range(H)]
    parts.append(s)
    parts.append(jnp.zeros((BE, 256 - F - H), jnp.float32))
    vals = jnp.concatenate(parts, axis=1)

    contrib = lax.dot_general(
        oh, vals, (((0,), (0,)), ((), ())),
        preferred_element_type=jnp.float32)
    num_ref[pl.ds(dbase, RW), :] = num_ref[pl.ds(dbase, RW), :] + contrib


def _segment_sum(hsrc, dstv, snode_pad, Apack, dmins):
    return pl.pallas_call(
        _seg_body,
        grid=(NBLK,),
        in_specs=[
            pl.BlockSpec((BE, F), lambda b: (b, 0)),
            pl.BlockSpec((BE, 1), lambda b: (b, 0)),
            pl.BlockSpec((NPAD, 16), lambda b: (0, 0)),
            pl.BlockSpec((F, 16), lambda b: (0, 0)),
            pl.BlockSpec(memory_space=pltpu.SMEM),
        ],
        out_specs=pl.BlockSpec((NPAD, 256), lambda b: (0, 0)),
        out_shape=jax.ShapeDtypeStruct((NPAD, 256), jnp.float32),
    )(hsrc, dstv, snode_pad, Apack, dmins)


def _pool_body(num_ref, batch_ref, bias_ref, gmp_ref, gap_ref,
               gmax_s, gsum_s, gcnt_s):
    b = pl.program_id(0)

    @pl.when(b == 0)
    def _():
        gmax_s[...] = jnp.full((G, F), -jnp.inf, jnp.float32)
        gsum_s[...] = jnp.zeros((G, F), jnp.float32)
        gcnt_s[...] = jnp.zeros((G, F), jnp.float32)

    nb = num_ref[...]
    hn_parts = [
        nb[:, hh * C:(hh + 1) * C] / (nb[:, F + hh:F + hh + 1] + 1e-16)
        for hh in range(H)
    ]
    hn = jnp.concatenate(hn_parts, axis=1) + bias_ref[...]
    hn = jnp.maximum(hn, 0.0)

    bid = batch_ref[...]
    ohg = (bid == lax.broadcasted_iota(jnp.int32, (BN, G), 1)
           ).astype(jnp.float32)
    gsum_s[...] += lax.dot_general(
        ohg, hn, (((0,), (0,)), ((), ())), preferred_element_type=jnp.float32)
    gcnt_s[...] += lax.dot_general(
        ohg, jnp.ones((BN, F), jnp.float32), (((0,), (0,)), ((), ())),
        preferred_element_type=jnp.float32)
    for g in range(G):
        m = jnp.max(jnp.where(bid == g, hn, -jnp.inf), axis=0, keepdims=True)
        gmax_s[g:g + 1, :] = jnp.maximum(gmax_s[g:g + 1, :], m)

    @pl.when(b == (N // BN) - 1)
    def _():
        gmp_ref[...] = gmax_s[...]
        gap_ref[...] = gsum_s[...] / jnp.maximum(gcnt_s[...], 1.0)


def _pool(num_full, batchv, biasv):
    return pl.pallas_call(
        _pool_body,
        grid=(N // BN,),
        in_specs=[
            pl.BlockSpec((BN, 256), lambda b: (b, 0)),
            pl.BlockSpec((BN, 1), lambda b: (b, 0)),
            pl.BlockSpec((1, F), lambda b: (0, 0)),
        ],
        out_specs=[
            pl.BlockSpec((G, F), lambda b: (0, 0)),
            pl.BlockSpec((G, F), lambda b: (0, 0)),
        ],
        out_shape=[
            jax.ShapeDtypeStruct((G, F), jnp.float32),
            jax.ShapeDtypeStruct((G, F), jnp.float32),
        ],
        scratch_shapes=[
            pltpu.VMEM((G, F), jnp.float32),
            pltpu.VMEM((G, F), jnp.float32),
            pltpu.VMEM((G, F), jnp.float32),
        ],
    )(num_full, batchv, biasv)


@jax.jit
def kernel(x, edge_index, batch, W, att_src, att_dst, bias):
    Apack = jnp.zeros((F, 16), jnp.float32)
    for hh in range(H):
        Apack = Apack.at[hh * C:(hh + 1) * C, hh].set(att_src[hh])
        Apack = Apack.at[hh * C:(hh + 1) * C, H + hh].set(att_dst[hh])

    h, snode = _project(x, W, Apack)

    loop = jnp.arange(N, dtype=jnp.int32)
    pad = E2 - (E + N)
    src_p = jnp.concatenate([edge_index[0], loop, jnp.zeros((pad,), jnp.int32)])
    dst_p = jnp.concatenate([edge_index[1], loop, jnp.full((pad,), N, jnp.int32)])
    order = jnp.argsort(dst_p)
    src_s = src_p[order]
    dst_s = dst_p[order]

    hsrc = _sc_gather(h, src_s)

    dstv = dst_s.reshape(E2, 1)
    dmins = dst_s.reshape(NBLK, BE)[:, 0:1]
    snode_pad = jnp.pad(snode, ((0, NPAD - N), (0, 0)))
    num_full = _segment_sum(hsrc, dstv, snode_pad, Apack, dmins)

    gmp, gap = _pool(num_full, batch.reshape(N, 1), bias.reshape(1, F))
    return jnp.concatenate([gmp, gap], axis=1)
